# baseline (speedup 1.0000x reference)
import jax
import jax.numpy as jnp
from jax import lax
from jax.experimental import pallas as pl
from jax.experimental.pallas import tpu as pltpu
from jax.experimental.pallas import tpu_sc as plsc

NC = 2
NS = 16
NW = NC * NS
GA = 128
NBUF = 2
L = 16


def _repack(src, dst, n):
    @pl.loop(0, n // 4)
    def _(q):
        for r in range(4):
            j = q * 4 + r
            dst[q, pl.ds(r * 32, L)] = src[j, pl.ds(0, L)]
            dst[q, pl.ds(r * 32 + L, L)] = src[j, pl.ds(L, L)]


def _body(idx_hbm, table_hbm, out_hbm, idx_v, vo, vg0, vg1, iq0, iq1,
          sem_g0, sem_g1, sem_s0, sem_s1, wc):
    ng = idx_hbm.shape[1] // GA
    v = table_hbm.shape[0]
    slab = (v // NS) // 8 * 8
    nch = slab // GA
    if nch % 2:
        nch -= 1
    tail_lo = slab - nch * GA
    last = v - (NS - 1) * slab
    tail_hi = last - nch * GA

    cid = lax.axis_index("c")
    sid = lax.axis_index("s")
    wid = sid * NC + cid
    sem_g = (sem_g0, sem_g1)
    sem_s = (sem_s0, sem_s1)
    vg = (vg0, vg1)
    iq = (iq0, iq1)

    pltpu.sync_copy(idx_hbm.at[wid], idx_v)

    base = sid * slab

    @pl.loop(0, nch // NBUF)
    def _slab(i):
        for b in range(NBUF):
            s = i * NBUF + b
            cbase = base + s * GA

            @pl.when(s >= NBUF)
            def _():
                pltpu.make_async_copy(
                    vg[b].at[pl.ds(0, GA // 4)],
                    wc.at[cid, pl.ds((cbase - NBUF * GA) // 4, GA // 4)],
                    sem_s[b]).wait()

            pltpu.async_copy(table_hbm.at[pl.ds(cbase, GA)], vo.at[b],
                             sem_g[b])
            pltpu.make_async_copy(table_hbm.at[pl.ds(cbase, GA)], vo.at[b],
                                  sem_g[b]).wait()
            _repack(vo.at[b], vg[b], GA)
            pltpu.async_copy(vg[b].at[pl.ds(0, GA // 4)],
                             wc.at[cid, pl.ds(cbase // 4, GA // 4)], sem_s[b])

    for b in range(NBUF):
        pltpu.make_async_copy(vg[b].at[pl.ds(0, GA // 4)],
                              wc.at[cid, pl.ds(0, GA // 4)], sem_s[b]).wait()

    tbase = base + nch * GA
    if tail_lo:
        @pl.when(sid < NS - 1)
        def _():
            pltpu.sync_copy(table_hbm.at[pl.ds(tbase, tail_lo)],
                            vo.at[0, pl.ds(0, tail_lo)])
            _repack(vo.at[0], vg[0], tail_lo)
            pltpu.sync_copy(vg[0].at[pl.ds(0, tail_lo // 4)],
                            wc.at[cid, pl.ds(tbase // 4, tail_lo // 4)])
    if tail_hi:
        @pl.when(sid == NS - 1)
        def _():
            pltpu.sync_copy(table_hbm.at[pl.ds(tbase, tail_hi)],
                            vo.at[0, pl.ds(0, tail_hi)])
            _repack(vo.at[0], vg[0], tail_hi)
            pltpu.sync_copy(vg[0].at[pl.ds(0, tail_hi // 4)],
                            wc.at[cid, pl.ds(tbase // 4, tail_hi // 4)])

    plsc.subcore_barrier()

    out_base = wid * ng * GA

    def _prep(h, b):
        for t in range(GA // L):
            toks = idx_v[pl.ds(h * GA + t * L, L)]
            iq[b][pl.ds(t * L, L)] = lax.shift_right_logical(toks, 2)

    def _extract(g, b):
        @pl.loop(0, GA // L)
        def _(t):
            toks = idx_v[pl.ds(g * GA + t * L, L)]
            for l in range(L):
                j = t * L + l
                off = (toks[l] & 3) * 32
                vo[b, j, pl.ds(0, L)] = vg[b][j, pl.ds(off, L)]
                vo[b, j, pl.ds(L, L)] = vg[b][j, pl.ds(off + L, L)]

    _prep(0, 0)
    pltpu.async_copy(wc.at[cid].at[iq0], vg0, sem_g0)

    @pl.loop(0, ng // NBUF)
    def _grp(i):
        for b in range(NBUF):
            g = i * NBUF + b
            gbase = out_base + g * GA

            @pl.when(g + 1 < ng)
            def _():
                _prep(g + 1, 1 - b)
                pltpu.async_copy(wc.at[cid].at[iq[1 - b]], vg[1 - b],
                                 sem_g[1 - b])

            pltpu.make_async_copy(wc.at[cid, pl.ds(0, GA)], vg[b],
                                  sem_g[b]).wait()

            @pl.when(g >= NBUF)
            def _():
                pltpu.make_async_copy(
                    vo.at[b], out_hbm.at[pl.ds(gbase - NBUF * GA, GA)],
                    sem_s[b]).wait()

            _extract(g, b)
            pltpu.async_copy(vo.at[b], out_hbm.at[pl.ds(gbase, GA)], sem_s[b])

    for b in range(NBUF):
        g = ng - NBUF + b
        pltpu.make_async_copy(vo.at[b],
                              out_hbm.at[pl.ds(out_base + g * GA, GA)],
                              sem_s[b]).wait()


def kernel(token_ids, weight):
    v, d = weight.shape
    total = 1
    for s in token_ids.shape:
        total *= s
    per_w = total // NW
    assert total == NW * per_w and per_w % GA == 0
    assert (per_w // GA) % NBUF == 0 and d == 32 and v % 4 == 0

    ids = token_ids.reshape(-1).astype(jnp.int32).reshape(NW, per_w)

    k = pl.kernel(
        _body,
        out_type=jax.ShapeDtypeStruct((total, d), jnp.float32),
        mesh=plsc.VectorSubcoreMesh(core_axis_name="c", subcore_axis_name="s"),
        compiler_params=pltpu.CompilerParams(use_tc_tiling_on_sc=True),
        scratch_types=[
            pltpu.VMEM((per_w,), jnp.int32),
            pltpu.VMEM((NBUF, GA, d), jnp.float32),
            pltpu.VMEM((GA, 4 * d), jnp.float32),
            pltpu.VMEM((GA, 4 * d), jnp.float32),
            pltpu.VMEM((GA,), jnp.int32),
            pltpu.VMEM((GA,), jnp.int32),
            pltpu.SemaphoreType.DMA,
            pltpu.SemaphoreType.DMA,
            pltpu.SemaphoreType.DMA,
            pltpu.SemaphoreType.DMA,
            pltpu.HBM((NC, v // 4, 4 * d), jnp.float32),
        ],
    )
    out = k(ids, weight)
    return out.reshape(*token_ids.shape, d)
